# baseline (speedup 1.0000x reference)
import functools

import jax
import jax.numpy as jnp
from jax import lax
from jax.experimental import pallas as pl
from jax.experimental.pallas import tpu as pltpu
from jax.experimental.pallas import tpu_sc as plsc

NC = 2
NS = 16
NW = NC * NS
M = 128
L = 16


@functools.partial(jax.jit, static_argnames=("n_steps", "nbt", "b1"))
def _sc_gather(weight, idx, n_steps, nbt, b1):
    V, D = weight.shape
    KH = D // 8

    mesh = plsc.VectorSubcoreMesh(core_axis_name="c", subcore_axis_name="s")

    @functools.partial(
        pl.kernel,
        out_type=jax.ShapeDtypeStruct((b1, KH, nbt, 8, M), jnp.float32),
        mesh=mesh,
        scratch_types=[
            pltpu.VMEM((n_steps, M), jnp.int32),
            [pltpu.VMEM((M, D), jnp.float32) for _ in range(2)],
            [pltpu.VMEM((KH, 8, M), jnp.float32) for _ in range(2)],
            pltpu.SemaphoreType.DMA,
            pltpu.SemaphoreType.DMA,
        ],
        compiler_params=pltpu.CompilerParams(
            use_tc_tiling_on_sc=False, needs_layout_passes=False),
    )
    def k(table_hbm, idx_hbm, out_hbm, idx_v, rows, ts, gsem, wsem):
        wid = lax.axis_index("s") * NC + lax.axis_index("c")
        pltpu.sync_copy(idx_hbm.at[wid], idx_v)
        cbase = wid * n_steps

        base = [lax.iota(jnp.int32, L) + L * kk for kk in range(M // L)]

        for b in range(2):
            pltpu.async_copy(table_hbm.at[idx_v.at[b]], rows[b], gsem)

        @pl.loop(0, n_steps, step=2)
        def _(i):
            for b in range(2):
                j = i + b
                pltpu.make_async_copy(table_hbm.at[pl.ds(0, M)], rows[b], gsem).wait()

                @pl.when(j >= 2)
                def _():
                    pltpu.make_async_copy(ts[b], out_hbm.at[0, :, 0], wsem).wait()

                for jj in range(D):
                    cj = jnp.full((L,), jj, jnp.int32)
                    for kk in range(M // L):
                        val = plsc.load_gather(rows[b], [base[kk], cj])
                        ts[b][jj // 8, jj % 8, pl.ds(L * kk, L)] = val

                c = cbase + j
                s = c // nbt
                bt = lax.rem(c, nbt)
                pltpu.async_copy(ts[b], out_hbm.at[s, :, bt], wsem)

                @pl.when(j + 2 < n_steps)
                def _():
                    pltpu.async_copy(table_hbm.at[idx_v.at[j + 2]], rows[b], gsem)

        for b in range(2):
            pltpu.make_async_copy(ts[b], out_hbm.at[0, :, 0], wsem).wait()

    return k(weight, idx)


def kernel(x, weight):
    B0, B1 = x.shape
    V, D = weight.shape
    nbt = B0 // M
    n_steps = B1 * nbt // NW
    xt = jnp.swapaxes(x, 0, 1).astype(jnp.int32)
    idx = xt.reshape(NW, n_steps, M)
    out5 = _sc_gather(weight, idx, n_steps, nbt, B1)
    return out5.transpose(2, 4, 0, 1, 3).reshape(B0, B1, D)


# speedup vs baseline: 2.5662x; 2.5662x over previous
import functools

import jax
import jax.numpy as jnp
from jax import lax
from jax.experimental import pallas as pl
from jax.experimental.pallas import tpu as pltpu
from jax.experimental.pallas import tpu_sc as plsc

NC = 2
NS = 16
NW = NC * NS
M = 128
L = 16


@functools.partial(jax.jit, static_argnames=("n_steps", "nbt", "b1"))
def _sc_gather(weight, idx, n_steps, nbt, b1):
    V, D = weight.shape
    KH = D // 8

    mesh = plsc.VectorSubcoreMesh(core_axis_name="c", subcore_axis_name="s")

    @functools.partial(
        pl.kernel,
        out_type=jax.ShapeDtypeStruct((b1, KH, nbt, 8, M), jnp.float32),
        mesh=mesh,
        scratch_types=[
            pltpu.VMEM((n_steps, M), jnp.int32),
            [pltpu.VMEM((M, D), jnp.float32) for _ in range(2)],
            [pltpu.VMEM((KH, 8, M + 1), jnp.float32) for _ in range(2)],
            pltpu.SemaphoreType.DMA,
            pltpu.SemaphoreType.DMA,
        ],
        compiler_params=pltpu.CompilerParams(
            use_tc_tiling_on_sc=False, needs_layout_passes=False,
            disable_bounds_checks=True),
    )
    def k(table_hbm, idx_hbm, out_hbm, idx_v, rows, ts, gsem, wsem):
        wid = lax.axis_index("s") * NC + lax.axis_index("c")
        pltpu.sync_copy(idx_hbm.at[wid], idx_v)
        cbase = wid * n_steps

        lane = lax.iota(jnp.int32, L)
        jt_idx = [2 * q + lax.shift_right_logical(lane, 3) for q in range(D // L)]
        jm_idx = lax.bitwise_and(lane, 7)

        for b in range(2):
            pltpu.async_copy(table_hbm.at[idx_v.at[b]], rows[b], gsem)

        @pl.loop(0, n_steps, step=2)
        def _(i):
            for b in range(2):
                j = i + b
                pltpu.make_async_copy(table_hbm.at[pl.ds(0, M)], rows[b], gsem).wait()

                @pl.when(j >= 2)
                def _():
                    pltpu.make_async_copy(ts[b].at[:, :, pl.ds(0, M)], out_hbm.at[0, :, 0], wsem).wait()

                @plsc.parallel_loop(0, M, unroll=4)
                def _(bm):
                    sb = jnp.full((L,), 1, jnp.int32) * bm
                    for q in range(D // L):
                        val = rows[b][bm, pl.ds(L * q, L)]
                        plsc.store_scatter(ts[b], [jt_idx[q], jm_idx, sb], val)

                c = cbase + j
                s = c // nbt
                bt = lax.rem(c, nbt)
                pltpu.async_copy(
                    ts[b].at[:, :, pl.ds(0, M)], out_hbm.at[s, :, bt], wsem)

                @pl.when(j + 2 < n_steps)
                def _():
                    pltpu.async_copy(table_hbm.at[idx_v.at[j + 2]], rows[b], gsem)

        for b in range(2):
            pltpu.make_async_copy(ts[b].at[:, :, pl.ds(0, M)], out_hbm.at[0, :, 0], wsem).wait()

    return k(weight, idx)


def kernel(x, weight):
    B0, B1 = x.shape
    V, D = weight.shape
    nbt = B0 // M
    n_steps = B1 * nbt // NW
    xt = jnp.swapaxes(x, 0, 1).astype(jnp.int32)
    idx = xt.reshape(NW, n_steps, M)
    out5 = _sc_gather(weight, idx, n_steps, nbt, B1)
    return out5.transpose(2, 4, 0, 1, 3).reshape(B0, B1, D)


# speedup vs baseline: 2.6832x; 1.0456x over previous
import functools

import jax
import jax.numpy as jnp
from jax import lax
from jax.experimental import pallas as pl
from jax.experimental.pallas import tpu as pltpu
from jax.experimental.pallas import tpu_sc as plsc

NC = 2
NS = 16
NW = NC * NS
M = 128
L = 16


@functools.partial(jax.jit, static_argnames=("n_steps", "nbt", "b1"))
def _sc_gather(weight, idx, n_steps, nbt, b1):
    V, D = weight.shape
    KH = D // 8

    mesh = plsc.VectorSubcoreMesh(core_axis_name="c", subcore_axis_name="s")

    @functools.partial(
        pl.kernel,
        out_type=jax.ShapeDtypeStruct((b1, KH, nbt, 8, M), jnp.float32),
        mesh=mesh,
        scratch_types=[
            pltpu.VMEM((n_steps, M), jnp.int32),
            [pltpu.VMEM((M, D), jnp.float32) for _ in range(4)],
            [pltpu.VMEM((KH, 8, M + 1), jnp.float32) for _ in range(2)],
            pltpu.SemaphoreType.DMA,
            pltpu.SemaphoreType.DMA,
        ],
        compiler_params=pltpu.CompilerParams(
            use_tc_tiling_on_sc=False, needs_layout_passes=False,
            disable_bounds_checks=True),
    )
    def k(table_hbm, idx_hbm, out_hbm, idx_v, rows, ts, gsem, wsem):
        wid = lax.axis_index("s") * NC + lax.axis_index("c")
        pltpu.sync_copy(idx_hbm.at[wid], idx_v)
        cbase = wid * n_steps

        lane = lax.iota(jnp.int32, L)
        jt_idx = [2 * q + lax.shift_right_logical(lane, 3) for q in range(D // L)]
        jm_idx = lax.bitwise_and(lane, 7)

        for b in range(3):
            pltpu.async_copy(table_hbm.at[idx_v.at[b]], rows[b], gsem)

        @pl.loop(0, n_steps, step=4)
        def _(i):
            for b in range(4):
                j = i + b
                tb = b % 2
                pltpu.make_async_copy(table_hbm.at[pl.ds(0, M)], rows[b], gsem).wait()

                @pl.when(j >= 2)
                def _():
                    pltpu.make_async_copy(ts[tb].at[:, :, pl.ds(0, M)], out_hbm.at[0, :, 0], wsem).wait()

                @pl.when(j + 3 < n_steps)
                def _():
                    pltpu.async_copy(table_hbm.at[idx_v.at[j + 3]],
                                     rows[(b - 1) % 4], gsem)

                @plsc.parallel_loop(0, M, unroll=8)
                def _(bm):
                    sb = jnp.full((L,), 1, jnp.int32) * bm
                    for q in range(D // L):
                        val = rows[b][bm, pl.ds(L * q, L)]
                        plsc.store_scatter(ts[tb], [jt_idx[q], jm_idx, sb], val)

                c = cbase + j
                s = c // nbt
                bt = lax.rem(c, nbt)
                pltpu.async_copy(
                    ts[tb].at[:, :, pl.ds(0, M)], out_hbm.at[s, :, bt], wsem)

        for b in range(2):
            pltpu.make_async_copy(ts[b].at[:, :, pl.ds(0, M)], out_hbm.at[0, :, 0], wsem).wait()

    return k(weight, idx)


def kernel(x, weight):
    B0, B1 = x.shape
    V, D = weight.shape
    nbt = B0 // M
    n_steps = B1 * nbt // NW
    xt = jnp.swapaxes(x, 0, 1).astype(jnp.int32)
    idx = xt.reshape(NW, n_steps, M)
    out5 = _sc_gather(weight, idx, n_steps, nbt, B1)
    return out5.transpose(2, 4, 0, 1, 3).reshape(B0, B1, D)
